# Initial kernel scaffold; baseline (speedup 1.0000x reference)
import jax
import jax.numpy as jnp
from jax.experimental import pallas as pl


def kernel(edge_val, edge_index):
    raise NotImplementedError("write your pallas kernel here")


# speedup vs baseline: 42.7489x; 42.7489x over previous
import functools

import jax
import jax.numpy as jnp
from jax import lax
from jax.experimental import pallas as pl
from jax.experimental.pallas import tpu as pltpu
from jax.experimental.pallas import tpu_sc as plsc

N_NODES = 100000
N_HEADS = 8

_NC = 2
_NS = 16
_NW = _NC * _NS

_B = 2048
_NVC = _B // 2

_ROWS_T = (N_NODES // _NS) // 8 * 8
_TAIL = N_NODES - _ROWS_T * _NS
_SUBS = (1560, 1560, 1560, 1568)
_SUBMAX = max(_SUBS)

_INTERPRET = False

_PARAMS = pltpu.CompilerParams(
    needs_layout_passes=False, use_tc_tiling_on_sc=False
)
_MESH = plsc.VectorSubcoreMesh(core_axis_name="c", subcore_axis_name="s")


def _halving_scale(m8):
    def cond(c):
        return jnp.any(c[0] > 10.0)

    def body(c):
        m, s = c
        h = m > 10.0
        return jnp.where(h, m * 0.5, m), jnp.where(h, s * 0.5, s)

    _, scale = lax.while_loop(cond, body, (m8, jnp.ones_like(m8)))
    return scale


def _chunk_bounds(wid, nch_total):
    first = wid * nch_total // _NW
    count = (wid + 1) * nch_total // _NW - first
    return first, count


def _sc_scatter(e_total, scaled):
    nch_total = e_total // _B

    out_type = [jax.ShapeDtypeStruct((_NC, N_NODES, 8), jnp.float32)]
    if not scaled:
        out_type.append(jax.ShapeDtypeStruct((_NC, _NS, 16), jnp.float32))

    @functools.partial(
        pl.kernel,
        out_type=out_type,
        mesh=_MESH,
        scratch_types=[
            pltpu.VMEM((_B, 8), jnp.float32),
            pltpu.VMEM((_B, 8), jnp.float32),
            pltpu.VMEM((_B,), jnp.int32),
            pltpu.VMEM((16,), jnp.float32),
            pltpu.VMEM_SHARED((N_NODES, 8), jnp.float32),
        ],
        compiler_params=_PARAMS,
        interpret=_INTERPRET,
    )
    def k2(ev_hbm, row_hbm, *args):
        if scaled:
            (scale_hbm, zeros_hbm, partials_hbm,
             vbuf, ubuf, idxbuf, sbuf, accum) = args
        else:
            (zeros_hbm, partials_hbm, max_hbm,
             vbuf, ubuf, idxbuf, sbuf, accum) = args
        c = lax.axis_index("c")
        s = lax.axis_index("s")
        wid = c * _NS + s

        @pl.when(s == 0)
        def _():
            pltpu.sync_copy(zeros_hbm, accum)

        if scaled:
            pltpu.sync_copy(scale_hbm, sbuf)
        plsc.subcore_barrier()

        sv = sbuf[...] if scaled else None
        first, count = _chunk_bounds(wid, nch_total)
        i16 = lax.iota(jnp.int32, 16)
        pat0, pat1 = i16 >> 3, i16 & 7

        def chunk(ci, macc):
            eb = pl.multiple_of((first + ci) * _B, _B)
            pltpu.sync_copy(ev_hbm.at[pl.ds(eb, _B)], vbuf)
            pltpu.sync_copy(row_hbm.at[pl.ds(eb, _B)], idxbuf)

            def vbody(k, m):
                i0, i1 = 2 * k + pat0, pat1
                v = plsc.load_gather(vbuf, [i0, i1])
                if scaled:
                    e = jnp.exp(v * sv)
                else:
                    e = jnp.exp(v)
                    m = jnp.maximum(m, v)
                plsc.store_scatter(ubuf, [i0, i1], e)
                return m

            macc = lax.fori_loop(0, _NVC, vbody, macc, unroll=4)
            pltpu.sync_copy(ubuf, accum.at[idxbuf], add=True)
            return macc

        macc = lax.fori_loop(
            0, count, chunk, jnp.full((16,), -jnp.inf, jnp.float32)
        )
        if not scaled:
            sbuf[...] = macc
            pltpu.sync_copy(sbuf, max_hbm.at[c, s])
        plsc.subcore_barrier()

        @pl.when(s == 0)
        def _():
            pltpu.sync_copy(accum, partials_hbm.at[c])

    return k2


def _sc_gather_div(e_total, scaled):
    nch_total = e_total // _B

    @functools.partial(
        pl.kernel,
        out_type=jax.ShapeDtypeStruct((e_total, 8), jnp.float32),
        mesh=_MESH,
        scratch_types=[
            pltpu.VMEM((_B, 8), jnp.float32),
            pltpu.VMEM((_B, 8), jnp.float32),
            pltpu.VMEM((_B,), jnp.int32),
            pltpu.VMEM((16,), jnp.float32),
            pltpu.VMEM((_SUBMAX, 8), jnp.float32),
            pltpu.VMEM((_SUBMAX, 8), jnp.float32),
            pltpu.VMEM_SHARED((N_NODES, 8), jnp.float32),
        ],
        compiler_params=_PARAMS,
        interpret=_INTERPRET,
    )
    def k3(ev_hbm, row_hbm, *args):
        if scaled:
            (scale_hbm, partials_hbm, out_hbm,
             vbuf, sums, idxbuf, sbuf, pa, pb, accum) = args
        else:
            (partials_hbm, out_hbm,
             vbuf, sums, idxbuf, sbuf, pa, pb, accum) = args
        c = lax.axis_index("c")
        s = lax.axis_index("s")
        wid = c * _NS + s
        i16 = lax.iota(jnp.int32, 16)
        pat0, pat1 = i16 >> 3, i16 & 7

        def combine(lo, rows):
            lo = pl.multiple_of(lo, 8)
            pltpu.sync_copy(partials_hbm.at[0, pl.ds(lo, rows)],
                            pa.at[pl.ds(0, rows)])
            pltpu.sync_copy(partials_hbm.at[1, pl.ds(lo, rows)],
                            pb.at[pl.ds(0, rows)])

            def abody(k, _):
                i0, i1 = 2 * k + pat0, pat1
                v = (plsc.load_gather(pa, [i0, i1])
                     + plsc.load_gather(pb, [i0, i1]))
                plsc.store_scatter(pa, [i0, i1], v)
                return 0

            lax.fori_loop(0, rows // 2, abody, 0, unroll=4)
            pltpu.sync_copy(pa.at[pl.ds(0, rows)], accum.at[pl.ds(lo, rows)])

        off = 0
        for sub in _SUBS:
            combine(s * _ROWS_T + off, sub)
            off += sub

        @pl.when(s == 0)
        def _():
            combine(_ROWS_T * _NS, _TAIL)

        if scaled:
            pltpu.sync_copy(scale_hbm, sbuf)
        plsc.subcore_barrier()

        sv = sbuf[...] if scaled else None
        first, count = _chunk_bounds(wid, nch_total)

        def chunk(ci, _):
            eb = pl.multiple_of((first + ci) * _B, _B)
            pltpu.sync_copy(ev_hbm.at[pl.ds(eb, _B)], vbuf)
            pltpu.sync_copy(row_hbm.at[pl.ds(eb, _B)], idxbuf)
            pltpu.sync_copy(accum.at[idxbuf], sums)

            def vbody(k, _):
                i0, i1 = 2 * k + pat0, pat1
                v = plsc.load_gather(vbuf, [i0, i1])
                ns = plsc.load_gather(sums, [i0, i1])
                e = jnp.exp(v * sv) if scaled else jnp.exp(v)
                plsc.store_scatter(vbuf, [i0, i1], e / ns)
                return 0

            lax.fori_loop(0, _NVC, vbody, 0, unroll=4)
            pltpu.sync_copy(vbuf, out_hbm.at[pl.ds(eb, _B)])
            return 0

        lax.fori_loop(0, count, chunk, 0)

    return k3


def kernel(edge_val, edge_index):
    e, h = edge_val.shape
    row = edge_index[0]
    zeros = jnp.zeros((N_NODES, 8), jnp.float32)

    partials, wmax = _sc_scatter(e, scaled=False)(edge_val, row, zeros)
    m8 = jnp.max(wmax.reshape(-1, 8), axis=0)

    def fast():
        return _sc_gather_div(e, scaled=False)(edge_val, row, partials)

    def slow():
        scale16 = jnp.tile(_halving_scale(m8), 2)
        p2 = _sc_scatter(e, scaled=True)(edge_val, row, scale16, zeros)[0]
        return _sc_gather_div(e, scaled=True)(edge_val, row, scale16, p2)

    return lax.cond(jnp.any(m8 > 10.0), slow, fast)


# speedup vs baseline: 162.1228x; 3.7924x over previous
import functools

import jax
import jax.numpy as jnp
from jax import lax
from jax.experimental import pallas as pl
from jax.experimental.pallas import tpu as pltpu
from jax.experimental.pallas import tpu_sc as plsc

N_NODES = 100000
N_HEADS = 8

_NC = 2
_NS = 16
_NW = _NC * _NS

_BLK = 16
_B = _BLK * 128

_ROWS_T = (N_NODES // _NS) // 8 * 8
_TAIL = N_NODES - _ROWS_T * _NS
_SUBS = (1560, 1560, 1560, 1568)
_SUBMAX = max(_SUBS)

_INTERPRET = False

_PARAMS = pltpu.CompilerParams(
    needs_layout_passes=False, use_tc_tiling_on_sc=False
)
_MESH = plsc.VectorSubcoreMesh(core_axis_name="c", subcore_axis_name="s")


def _halving_scale(m8):
    def cond(c):
        return jnp.any(c[0] > 10.0)

    def body(c):
        m, s = c
        h = m > 10.0
        return jnp.where(h, m * 0.5, m), jnp.where(h, s * 0.5, s)

    _, scale = lax.while_loop(cond, body, (m8, jnp.ones_like(m8)))
    return scale


def _chunk_bounds(wid, nch_total):
    first = wid * nch_total // _NW
    count = (wid + 1) * nch_total // _NW - first
    return first, count


def _copy_indices(rbuf, idxbuf):
    def ibody(j, _):
        idxbuf[pl.ds(j * 16, 16)] = rbuf[j >> 3, pl.ds((j & 7) * 16, 16)]
        return 0

    lax.fori_loop(0, _B // 16, ibody, 0, unroll=8)


def _sc_scatter(nb, scaled):
    nch_total = nb // _BLK

    out_type = [jax.ShapeDtypeStruct((_NC, N_NODES, 8), jnp.float32)]
    if not scaled:
        out_type.append(jax.ShapeDtypeStruct((_NC, _NS, 16), jnp.float32))

    @functools.partial(
        pl.kernel,
        out_type=out_type,
        mesh=_MESH,
        scratch_types=[
            pltpu.VMEM((_BLK, 8, 128), jnp.float32),
            pltpu.VMEM((_B, 8), jnp.float32),
            pltpu.VMEM((_BLK, 128), jnp.int32),
            pltpu.VMEM((_B,), jnp.int32),
            pltpu.VMEM((16,), jnp.float32),
            pltpu.VMEM_SHARED((N_NODES, 8), jnp.float32),
        ],
        compiler_params=_PARAMS,
        interpret=_INTERPRET,
    )
    def k2(ev_hbm, row_hbm, *args):
        if scaled:
            (scale_hbm, zeros_hbm, partials_hbm,
             vbuf, ubuf, rbuf, idxbuf, sbuf, accum) = args
        else:
            (zeros_hbm, partials_hbm, max_hbm,
             vbuf, ubuf, rbuf, idxbuf, sbuf, accum) = args
        c = lax.axis_index("c")
        s = lax.axis_index("s")
        wid = c * _NS + s

        @pl.when(s == 0)
        def _():
            pltpu.sync_copy(zeros_hbm, accum)

        if scaled:
            pltpu.sync_copy(scale_hbm, sbuf)
        plsc.subcore_barrier()

        first, count = _chunk_bounds(wid, nch_total)
        i16 = lax.iota(jnp.int32, 16)
        if scaled:
            svs = [
                plsc.load_gather(sbuf, [jnp.full((16,), h, jnp.int32)])
                for h in range(8)
            ]

        def chunk(ci, macc):
            eb = pl.multiple_of((first + ci) * _BLK, _BLK)
            pltpu.sync_copy(ev_hbm.at[pl.ds(eb, _BLK)], vbuf)
            pltpu.sync_copy(row_hbm.at[pl.ds(eb, _BLK)], rbuf)
            _copy_indices(rbuf, idxbuf)

            for h in range(8):
                i1 = jnp.full((16,), h, jnp.int32)

                def vbody(t, m):
                    v = vbuf[t >> 3, h, pl.ds((t & 7) * 16, 16)]
                    if scaled:
                        e = jnp.exp(v * svs[h])
                    else:
                        e = jnp.exp(v)
                        m = jnp.maximum(m, v)
                    plsc.store_scatter(ubuf, [t * 16 + i16, i1], e)
                    return m

                macc = lax.fori_loop(0, _B // 16, vbody, macc, unroll=8)

            pltpu.sync_copy(ubuf, accum.at[idxbuf], add=True)
            return macc

        macc = lax.fori_loop(
            0, count, chunk, jnp.full((16,), -jnp.inf, jnp.float32)
        )
        if not scaled:
            sbuf[...] = macc
            pltpu.sync_copy(sbuf, max_hbm.at[c, s])
        plsc.subcore_barrier()

        @pl.when(s == 0)
        def _():
            pltpu.sync_copy(accum, partials_hbm.at[c])

    return k2


def _sc_gather_div(nb):
    nch_total = nb // _BLK

    @functools.partial(
        pl.kernel,
        out_type=jax.ShapeDtypeStruct((nb, 8, 128), jnp.float32),
        mesh=_MESH,
        scratch_types=[
            pltpu.VMEM((_BLK, 8, 128), jnp.float32),
            pltpu.VMEM((_B, 8), jnp.float32),
            pltpu.VMEM((_BLK, 128), jnp.int32),
            pltpu.VMEM((_B,), jnp.int32),
            pltpu.VMEM((16,), jnp.float32),
            pltpu.VMEM((_SUBMAX, 8), jnp.float32),
            pltpu.VMEM((_SUBMAX, 8), jnp.float32),
            pltpu.VMEM_SHARED((N_NODES, 8), jnp.float32),
        ],
        compiler_params=_PARAMS,
        interpret=_INTERPRET,
    )
    def k3(ev_hbm, row_hbm, scale_hbm, partials_hbm, out_hbm, vbuf, sums,
           rbuf, idxbuf, sbuf, pa, pb, accum):
        c = lax.axis_index("c")
        s = lax.axis_index("s")
        wid = c * _NS + s
        i16 = lax.iota(jnp.int32, 16)
        pat0, pat1 = i16 >> 3, i16 & 7

        def combine(lo, rows):
            lo = pl.multiple_of(lo, 8)
            pltpu.sync_copy(partials_hbm.at[0, pl.ds(lo, rows)],
                            pa.at[pl.ds(0, rows)])
            pltpu.sync_copy(partials_hbm.at[1, pl.ds(lo, rows)],
                            pb.at[pl.ds(0, rows)])

            def abody(k, _):
                i0, i1 = 2 * k + pat0, pat1
                v = (plsc.load_gather(pa, [i0, i1])
                     + plsc.load_gather(pb, [i0, i1]))
                plsc.store_scatter(pa, [i0, i1], v)
                return 0

            lax.fori_loop(0, rows // 2, abody, 0, unroll=8)
            pltpu.sync_copy(pa.at[pl.ds(0, rows)], accum.at[pl.ds(lo, rows)])

        off = 0
        for sub in _SUBS:
            combine(s * _ROWS_T + off, sub)
            off += sub

        @pl.when(s == 0)
        def _():
            combine(_ROWS_T * _NS, _TAIL)

        pltpu.sync_copy(scale_hbm, sbuf)
        plsc.subcore_barrier()

        first, count = _chunk_bounds(wid, nch_total)
        svs = [
            plsc.load_gather(sbuf, [jnp.full((16,), h, jnp.int32)])
            for h in range(8)
        ]

        def chunk(ci, _):
            eb = pl.multiple_of((first + ci) * _BLK, _BLK)
            pltpu.sync_copy(ev_hbm.at[pl.ds(eb, _BLK)], vbuf)
            pltpu.sync_copy(row_hbm.at[pl.ds(eb, _BLK)], rbuf)
            _copy_indices(rbuf, idxbuf)
            pltpu.sync_copy(accum.at[idxbuf], sums)

            for h in range(8):
                i1 = jnp.full((16,), h, jnp.int32)

                def vbody(t, _):
                    sl = pl.ds((t & 7) * 16, 16)
                    v = vbuf[t >> 3, h, sl]
                    ns = plsc.load_gather(sums, [t * 16 + i16, i1])
                    vbuf[t >> 3, h, sl] = jnp.exp(v * svs[h]) / ns
                    return 0

                lax.fori_loop(0, _B // 16, vbody, 0, unroll=8)

            pltpu.sync_copy(vbuf, out_hbm.at[pl.ds(eb, _BLK)])
            return 0

        lax.fori_loop(0, count, chunk, 0)

    return k3


def kernel(edge_val, edge_index):
    e, h = edge_val.shape
    nb = e // 128
    ev3 = edge_val.reshape(nb, 128, h).transpose(0, 2, 1)
    row2 = edge_index[0].reshape(nb, 128)
    zeros = jnp.zeros((N_NODES, 8), jnp.float32)

    partials, wmax = _sc_scatter(nb, scaled=False)(ev3, row2, zeros)
    m8 = jnp.max(wmax.reshape(-1, 8), axis=0)
    scale16 = jnp.tile(_halving_scale(m8), 2)

    partials = lax.cond(
        jnp.any(m8 > 10.0),
        lambda: _sc_scatter(nb, scaled=True)(ev3, row2, scale16, zeros)[0],
        lambda: partials,
    )
    out3 = _sc_gather_div(nb)(ev3, row2, scale16, partials)
    return out3.transpose(0, 2, 1).reshape(e, h)


# speedup vs baseline: 162.1615x; 1.0002x over previous
import functools

import jax
import jax.numpy as jnp
from jax import lax
from jax.experimental import pallas as pl
from jax.experimental.pallas import tpu as pltpu
from jax.experimental.pallas import tpu_sc as plsc

N_NODES = 100000
N_HEADS = 8

_NC = 2
_NS = 16
_NW = _NC * _NS

_BLK2 = 16
_BLK3 = 8

_ROWS_T = (N_NODES // _NS) // 8 * 8
_TAIL = N_NODES - _ROWS_T * _NS
_SUBS = (776,) * 7 + (816,)
_SUBMAX = max(_SUBS)

_INTERPRET = False

_PARAMS = pltpu.CompilerParams(
    needs_layout_passes=False, use_tc_tiling_on_sc=False
)
_MESH = plsc.VectorSubcoreMesh(core_axis_name="c", subcore_axis_name="s")


def _halving_scale(m8):
    def cond(c):
        return jnp.any(c[0] > 10.0)

    def body(c):
        m, s = c
        h = m > 10.0
        return jnp.where(h, m * 0.5, m), jnp.where(h, s * 0.5, s)

    _, scale = lax.while_loop(cond, body, (m8, jnp.ones_like(m8)))
    return scale


def _chunk_bounds(wid, nch_total):
    first = wid * nch_total // _NW
    count = (wid + 1) * nch_total // _NW - first
    return first, count


def _copy_indices(rbuf, idxbuf, nblk):
    def ibody(j, _):
        idxbuf[pl.ds(j * 16, 16)] = rbuf[j >> 3, pl.ds((j & 7) * 16, 16)]
        return 0

    lax.fori_loop(0, nblk * 8, ibody, 0, unroll=8)


def _sc_scatter(nb, scaled):
    nch_total = nb // _BLK2
    nch_max = -(-nch_total // _NW)
    assert nch_max % 2 == 0
    b = _BLK2 * 128

    out_type = [jax.ShapeDtypeStruct((_NC, N_NODES, 8), jnp.float32)]
    if not scaled:
        out_type.append(jax.ShapeDtypeStruct((_NC, _NS, 16), jnp.float32))

    @functools.partial(
        pl.kernel,
        out_type=out_type,
        mesh=_MESH,
        scratch_types=[
            pltpu.VMEM((_BLK2, 8, 128), jnp.float32),
            pltpu.VMEM((_BLK2, 128), jnp.int32),
            pltpu.VMEM((b, 8), jnp.float32),
            pltpu.VMEM((b, 8), jnp.float32),
            pltpu.VMEM((b,), jnp.int32),
            pltpu.VMEM((b,), jnp.int32),
            pltpu.VMEM((16,), jnp.float32),
            pltpu.VMEM_SHARED((N_NODES, 8), jnp.float32),
            pltpu.SemaphoreType.DMA,
            pltpu.SemaphoreType.DMA,
        ],
        compiler_params=_PARAMS,
        interpret=_INTERPRET,
    )
    def k2(ev_hbm, row_hbm, *args):
        if scaled:
            (scale_hbm, zeros_hbm, partials_hbm, vbuf, rbuf, ubuf0, ubuf1,
             idx0, idx1, sbuf, accum, sem0, sem1) = args
        else:
            (zeros_hbm, partials_hbm, max_hbm, vbuf, rbuf, ubuf0, ubuf1,
             idx0, idx1, sbuf, accum, sem0, sem1) = args
        ubufs, idxs, sems = (ubuf0, ubuf1), (idx0, idx1), (sem0, sem1)
        c = lax.axis_index("c")
        s = lax.axis_index("s")
        wid = c * _NS + s

        @pl.when(s == 0)
        def _():
            pltpu.sync_copy(zeros_hbm, accum)

        if scaled:
            pltpu.sync_copy(scale_hbm, sbuf)
        plsc.subcore_barrier()

        first, count = _chunk_bounds(wid, nch_total)
        i16 = lax.iota(jnp.int32, 16)
        if scaled:
            svs = [
                plsc.load_gather(sbuf, [jnp.full((16,), h, jnp.int32)])
                for h in range(8)
            ]

        def step(st, macc):
            for p in (0, 1):
                ci = 2 * st + p
                cidx = jnp.minimum(first + ci, nch_total - 1)
                eb = pl.multiple_of(cidx * _BLK2, 8)
                pltpu.sync_copy(ev_hbm.at[pl.ds(eb, _BLK2)], vbuf)
                pltpu.sync_copy(row_hbm.at[pl.ds(eb, _BLK2)], rbuf)

                @pl.when(ci >= 2)
                def _():
                    pltpu.make_async_copy(
                        ubufs[p], accum.at[idxs[p]], sems[p]
                    ).wait()

                _copy_indices(rbuf, idxs[p], _BLK2)

                for h in range(8):
                    i1 = jnp.full((16,), h, jnp.int32)

                    def vbody(t, m):
                        v = vbuf[t >> 3, h, pl.ds((t & 7) * 16, 16)]
                        if scaled:
                            e = jnp.exp(v * svs[h])
                        else:
                            e = jnp.exp(v)
                            m = jnp.maximum(m, v)
                        plsc.store_scatter(
                            ubufs[p], [t * 16 + i16, i1], e
                        )
                        return m

                    macc = lax.fori_loop(0, b // 16, vbody, macc, unroll=8)

                @pl.when(ci < count)
                def _():
                    pltpu.async_copy(
                        ubufs[p], accum.at[idxs[p]], sems[p], add=True
                    )

            return macc

        macc = lax.fori_loop(
            0, nch_max // 2, step, jnp.full((16,), -jnp.inf, jnp.float32)
        )
        for p in (0, 1):
            @pl.when(nch_max - 2 + p < count)
            def _():
                pltpu.make_async_copy(
                    ubufs[p], accum.at[idxs[p]], sems[p]
                ).wait()

        if not scaled:
            sbuf[...] = macc
            pltpu.sync_copy(sbuf, max_hbm.at[c, s])
        plsc.subcore_barrier()

        @pl.when(s == 0)
        def _():
            pltpu.sync_copy(accum, partials_hbm.at[c])

    return k2


def _sc_gather_div(nb):
    nch_total = nb // _BLK3
    nch_max = -(-nch_total // _NW)
    b = _BLK3 * 128

    @functools.partial(
        pl.kernel,
        out_type=jax.ShapeDtypeStruct((nb, 8, 128), jnp.float32),
        mesh=_MESH,
        scratch_types=[
            pltpu.VMEM((_BLK3, 8, 128), jnp.float32),
            pltpu.VMEM((_BLK3, 8, 128), jnp.float32),
            pltpu.VMEM((_BLK3, 128), jnp.int32),
            pltpu.VMEM((b, 8), jnp.float32),
            pltpu.VMEM((b, 8), jnp.float32),
            pltpu.VMEM((b,), jnp.int32),
            pltpu.VMEM((b,), jnp.int32),
            pltpu.VMEM((16,), jnp.float32),
            pltpu.VMEM((_SUBMAX, 8), jnp.float32),
            pltpu.VMEM((_SUBMAX, 8), jnp.float32),
            pltpu.VMEM_SHARED((N_NODES, 8), jnp.float32),
            pltpu.SemaphoreType.DMA,
            pltpu.SemaphoreType.DMA,
            pltpu.SemaphoreType.DMA,
            pltpu.SemaphoreType.DMA,
        ],
        compiler_params=_PARAMS,
        interpret=_INTERPRET,
    )
    def k3(ev_hbm, row_hbm, scale_hbm, partials_hbm, out_hbm, vbuf0, vbuf1,
           rbuf, sums0, sums1, idx0, idx1, sbuf, pa, pb, accum,
           semg0, semg1, semo0, semo1):
        vbufs, sums, idxs = (vbuf0, vbuf1), (sums0, sums1), (idx0, idx1)
        semg, semo = (semg0, semg1), (semo0, semo1)
        c = lax.axis_index("c")
        s = lax.axis_index("s")
        wid = c * _NS + s
        i16 = lax.iota(jnp.int32, 16)
        pat0, pat1 = i16 >> 3, i16 & 7

        def combine(lo, rows):
            lo = pl.multiple_of(lo, 8)
            pltpu.sync_copy(partials_hbm.at[0, pl.ds(lo, rows)],
                            pa.at[pl.ds(0, rows)])
            pltpu.sync_copy(partials_hbm.at[1, pl.ds(lo, rows)],
                            pb.at[pl.ds(0, rows)])

            def abody(k, _):
                i0, i1 = 2 * k + pat0, pat1
                v = (plsc.load_gather(pa, [i0, i1])
                     + plsc.load_gather(pb, [i0, i1]))
                plsc.store_scatter(pa, [i0, i1], v)
                return 0

            lax.fori_loop(0, rows // 2, abody, 0, unroll=8)
            pltpu.sync_copy(pa.at[pl.ds(0, rows)], accum.at[pl.ds(lo, rows)])

        off = 0
        for sub in _SUBS:
            combine(s * _ROWS_T + off, sub)
            off += sub

        @pl.when(s == 0)
        def _():
            combine(_ROWS_T * _NS, _TAIL)

        pltpu.sync_copy(scale_hbm, sbuf)
        plsc.subcore_barrier()

        first, count = _chunk_bounds(wid, nch_total)
        svs = [
            plsc.load_gather(sbuf, [jnp.full((16,), h, jnp.int32)])
            for h in range(8)
        ]

        def chunk_eb(ci):
            cidx = jnp.minimum(first + ci, nch_total - 1)
            return pl.multiple_of(cidx * _BLK3, 8)

        def stage_a(ci, p):
            eb = chunk_eb(ci)

            @pl.when(ci >= 2)
            def _():
                pltpu.make_async_copy(
                    vbufs[p], out_hbm.at[pl.ds(chunk_eb(ci - 2), _BLK3)],
                    semo[p],
                ).wait()

            pltpu.sync_copy(ev_hbm.at[pl.ds(eb, _BLK3)], vbufs[p])
            pltpu.sync_copy(row_hbm.at[pl.ds(eb, _BLK3)], rbuf)
            _copy_indices(rbuf, idxs[p], _BLK3)
            pltpu.async_copy(accum.at[idxs[p]], sums[p], semg[p])

        def stage_b(ci, p):
            pltpu.make_async_copy(accum.at[idxs[p]], sums[p], semg[p]).wait()

            for h in range(8):
                i1 = jnp.full((16,), h, jnp.int32)

                def vbody(t, _):
                    sl = pl.ds((t & 7) * 16, 16)
                    v = vbufs[p][t >> 3, h, sl]
                    ns = plsc.load_gather(sums[p], [t * 16 + i16, i1])
                    vbufs[p][t >> 3, h, sl] = jnp.exp(v * svs[h]) / ns
                    return 0

                lax.fori_loop(0, b // 16, vbody, 0, unroll=8)

            @pl.when(ci < count)
            def _():
                pltpu.async_copy(
                    vbufs[p], out_hbm.at[pl.ds(chunk_eb(ci), _BLK3)], semo[p]
                )

        stage_a(jnp.int32(0), 0)

        def step(st, _):
            for p in (0, 1):
                ci = 2 * st + p
                stage_a(ci + 1, 1 - p)
                stage_b(ci, p)
            return 0

        lax.fori_loop(0, nch_max // 2, step, 0)
        pltpu.make_async_copy(accum.at[idxs[0]], sums[0], semg[0]).wait()

        @pl.when(nch_max - 1 < count)
        def _():
            pltpu.make_async_copy(
                vbufs[1], out_hbm.at[pl.ds(chunk_eb(0), _BLK3)], semo[1]
            ).wait()

    return k3


def kernel(edge_val, edge_index):
    e, h = edge_val.shape
    nb = e // 128
    ev3 = edge_val.reshape(nb, 128, h).transpose(0, 2, 1)
    row2 = edge_index[0].reshape(nb, 128)
    zeros = jnp.zeros((N_NODES, 8), jnp.float32)

    partials, wmax = _sc_scatter(nb, scaled=False)(ev3, row2, zeros)
    m8 = jnp.max(wmax.reshape(-1, 8), axis=0)
    scale16 = jnp.tile(_halving_scale(m8), 2)

    partials = lax.cond(
        jnp.any(m8 > 10.0),
        lambda: _sc_scatter(nb, scaled=True)(ev3, row2, scale16, zeros)[0],
        lambda: partials,
    )
    out3 = _sc_gather_div(nb)(ev3, row2, scale16, partials)
    return out3.transpose(0, 2, 1).reshape(e, h)


# speedup vs baseline: 197.3622x; 1.2171x over previous
import functools

import jax
import jax.numpy as jnp
from jax import lax
from jax.experimental import pallas as pl
from jax.experimental.pallas import tpu as pltpu
from jax.experimental.pallas import tpu_sc as plsc

N_NODES = 100000
N_HEADS = 8

_NC = 2
_NS = 16
_NW = _NC * _NS

_BLK2 = 16
_BLK3 = 8

_ROWS_T = (N_NODES // _NS) // 8 * 8
_TAIL = N_NODES - _ROWS_T * _NS
_SUBS = (776,) * 7 + (816,)
_SUBMAX = max(_SUBS)

_INTERPRET = False

_PARAMS = pltpu.CompilerParams(
    needs_layout_passes=False, use_tc_tiling_on_sc=False
)
_MESH = plsc.VectorSubcoreMesh(core_axis_name="c", subcore_axis_name="s")


def _halving_scale(m8):
    def cond(c):
        return jnp.any(c[0] > 10.0)

    def body(c):
        m, s = c
        h = m > 10.0
        return jnp.where(h, m * 0.5, m), jnp.where(h, s * 0.5, s)

    _, scale = lax.while_loop(cond, body, (m8, jnp.ones_like(m8)))
    return scale


def _chunk_bounds(wid, nch_total):
    first = wid * nch_total // _NW
    count = (wid + 1) * nch_total // _NW - first
    return first, count


def _copy_indices(rbuf, idxbuf, nblk):
    def ibody(j, _):
        idxbuf[pl.ds(j * 16, 16)] = rbuf[j >> 3, pl.ds((j & 7) * 16, 16)]
        return 0

    lax.fori_loop(0, nblk * 8, ibody, 0, unroll=8)


def _sc_scatter(nb, scaled):
    nch_total = nb // _BLK2
    nch_max = -(-nch_total // _NW)
    assert nch_max % 2 == 0
    b = _BLK2 * 128

    out_type = [jax.ShapeDtypeStruct((_NC, N_NODES, 8), jnp.float32)]
    if not scaled:
        out_type.append(jax.ShapeDtypeStruct((_NC, _NS, 16), jnp.float32))

    @functools.partial(
        pl.kernel,
        out_type=out_type,
        mesh=_MESH,
        scratch_types=[
            pltpu.VMEM((_BLK2, 8, 128), jnp.float32),
            pltpu.VMEM((_BLK2, 128), jnp.int32),
            pltpu.VMEM((b, 8), jnp.float32),
            pltpu.VMEM((b, 8), jnp.float32),
            pltpu.VMEM((b,), jnp.int32),
            pltpu.VMEM((b,), jnp.int32),
            pltpu.VMEM((16,), jnp.float32),
            pltpu.VMEM_SHARED((N_NODES, 8), jnp.float32),
            pltpu.SemaphoreType.DMA,
            pltpu.SemaphoreType.DMA,
        ],
        compiler_params=_PARAMS,
        interpret=_INTERPRET,
    )
    def k2(ev_hbm, row_hbm, *args):
        if scaled:
            (scale_hbm, zeros_hbm, partials_hbm, vbuf, rbuf, ubuf0, ubuf1,
             idx0, idx1, sbuf, accum, sem0, sem1) = args
        else:
            (zeros_hbm, partials_hbm, max_hbm, vbuf, rbuf, ubuf0, ubuf1,
             idx0, idx1, sbuf, accum, sem0, sem1) = args
        ubufs, idxs, sems = (ubuf0, ubuf1), (idx0, idx1), (sem0, sem1)
        c = lax.axis_index("c")
        s = lax.axis_index("s")
        wid = c * _NS + s

        @pl.when(s == 0)
        def _():
            pltpu.sync_copy(zeros_hbm, accum)

        if scaled:
            pltpu.sync_copy(scale_hbm, sbuf)
        plsc.subcore_barrier()

        first, count = _chunk_bounds(wid, nch_total)
        i16 = lax.iota(jnp.int32, 16)
        if scaled:
            svs = [
                plsc.load_gather(sbuf, [jnp.full((16,), h, jnp.int32)])
                for h in range(8)
            ]

        def step(st, macc):
            for p in (0, 1):
                ci = 2 * st + p
                cidx = jnp.minimum(first + ci, nch_total - 1)
                eb = pl.multiple_of(cidx * _BLK2, 8)
                pltpu.sync_copy(ev_hbm.at[pl.ds(eb, _BLK2)], vbuf)
                pltpu.sync_copy(row_hbm.at[pl.ds(eb, _BLK2)], rbuf)

                @pl.when(ci >= 2)
                def _():
                    pltpu.make_async_copy(
                        ubufs[p], accum.at[idxs[p]], sems[p]
                    ).wait()

                _copy_indices(rbuf, idxs[p], _BLK2)

                for h in range(8):
                    i1 = jnp.full((16,), h, jnp.int32)

                    def vbody(t, carry):
                        m, i0 = carry
                        v = vbuf[t >> 3, h, pl.ds((t & 7) * 16, 16)]
                        if scaled:
                            e = jnp.exp(v * svs[h])
                        else:
                            e = jnp.exp(v)
                            m = jnp.maximum(m, v)
                        plsc.store_scatter(ubufs[p], [i0, i1], e)
                        return m, i0 + 16

                    macc, _ = lax.fori_loop(
                        0, b // 16, vbody, (macc, i16), unroll=8
                    )

                @pl.when(ci < count)
                def _():
                    pltpu.async_copy(
                        ubufs[p], accum.at[idxs[p]], sems[p], add=True
                    )

            return macc

        macc = lax.fori_loop(
            0, nch_max // 2, step, jnp.full((16,), -jnp.inf, jnp.float32)
        )
        for p in (0, 1):
            @pl.when(nch_max - 2 + p < count)
            def _():
                pltpu.make_async_copy(
                    ubufs[p], accum.at[idxs[p]], sems[p]
                ).wait()

        if not scaled:
            sbuf[...] = macc
            pltpu.sync_copy(sbuf, max_hbm.at[c, s])
        plsc.subcore_barrier()

        @pl.when(s == 0)
        def _():
            pltpu.sync_copy(accum, partials_hbm.at[c])

    return k2


def _sc_gather_div(nb):
    nch_total = nb // _BLK3
    nch_max = -(-nch_total // _NW)
    b = _BLK3 * 128

    @functools.partial(
        pl.kernel,
        out_type=jax.ShapeDtypeStruct((nb, 8, 128), jnp.float32),
        mesh=_MESH,
        scratch_types=[
            pltpu.VMEM((_BLK3, 8, 128), jnp.float32),
            pltpu.VMEM((_BLK3, 8, 128), jnp.float32),
            pltpu.VMEM((_BLK3, 128), jnp.int32),
            pltpu.VMEM((b, 8), jnp.float32),
            pltpu.VMEM((b, 8), jnp.float32),
            pltpu.VMEM((b,), jnp.int32),
            pltpu.VMEM((b,), jnp.int32),
            pltpu.VMEM((16,), jnp.float32),
            pltpu.VMEM((_SUBMAX, 8), jnp.float32),
            pltpu.VMEM((_SUBMAX, 8), jnp.float32),
            pltpu.VMEM_SHARED((N_NODES, 8), jnp.float32),
            pltpu.SemaphoreType.DMA,
            pltpu.SemaphoreType.DMA,
            pltpu.SemaphoreType.DMA,
            pltpu.SemaphoreType.DMA,
        ],
        compiler_params=_PARAMS,
        interpret=_INTERPRET,
    )
    def k3(ev_hbm, row_hbm, scale_hbm, partials_hbm, out_hbm, vbuf0, vbuf1,
           rbuf, sums0, sums1, idx0, idx1, sbuf, pa, pb, accum,
           semg0, semg1, semo0, semo1):
        vbufs, sums, idxs = (vbuf0, vbuf1), (sums0, sums1), (idx0, idx1)
        semg, semo = (semg0, semg1), (semo0, semo1)
        c = lax.axis_index("c")
        s = lax.axis_index("s")
        wid = c * _NS + s
        i16 = lax.iota(jnp.int32, 16)
        pat0, pat1 = i16 >> 3, i16 & 7

        def combine(lo, rows):
            lo = pl.multiple_of(lo, 8)
            pltpu.sync_copy(partials_hbm.at[0, pl.ds(lo, rows)],
                            pa.at[pl.ds(0, rows)])
            pltpu.sync_copy(partials_hbm.at[1, pl.ds(lo, rows)],
                            pb.at[pl.ds(0, rows)])

            def abody(k, carry):
                i0 = carry
                v = (plsc.load_gather(pa, [i0, pat1])
                     + plsc.load_gather(pb, [i0, pat1]))
                plsc.store_scatter(pa, [i0, pat1], 1.0 / v)
                return i0 + 2

            lax.fori_loop(0, rows // 2, abody, pat0, unroll=8)
            pltpu.sync_copy(pa.at[pl.ds(0, rows)], accum.at[pl.ds(lo, rows)])

        off = 0
        for sub in _SUBS:
            combine(s * _ROWS_T + off, sub)
            off += sub

        @pl.when(s == 0)
        def _():
            combine(_ROWS_T * _NS, _TAIL)

        pltpu.sync_copy(scale_hbm, sbuf)
        plsc.subcore_barrier()

        first, count = _chunk_bounds(wid, nch_total)
        svs = [
            plsc.load_gather(sbuf, [jnp.full((16,), h, jnp.int32)])
            for h in range(8)
        ]

        def chunk_eb(ci):
            cidx = jnp.minimum(first + ci, nch_total - 1)
            return pl.multiple_of(cidx * _BLK3, 8)

        def stage_a(ci, p):
            eb = chunk_eb(ci)

            @pl.when(ci >= 2)
            def _():
                pltpu.make_async_copy(
                    vbufs[p], out_hbm.at[pl.ds(chunk_eb(ci - 2), _BLK3)],
                    semo[p],
                ).wait()

            pltpu.sync_copy(ev_hbm.at[pl.ds(eb, _BLK3)], vbufs[p])
            pltpu.sync_copy(row_hbm.at[pl.ds(eb, _BLK3)], rbuf)
            _copy_indices(rbuf, idxs[p], _BLK3)
            pltpu.async_copy(accum.at[idxs[p]], sums[p], semg[p])

        def stage_b(ci, p):
            pltpu.make_async_copy(accum.at[idxs[p]], sums[p], semg[p]).wait()

            for h in range(8):
                i1 = jnp.full((16,), h, jnp.int32)

                def vbody(t, i0):
                    sl = pl.ds((t & 7) * 16, 16)
                    v = vbufs[p][t >> 3, h, sl]
                    ns = plsc.load_gather(sums[p], [i0, i1])
                    vbufs[p][t >> 3, h, sl] = jnp.exp(v * svs[h]) * ns
                    return i0 + 16

                lax.fori_loop(0, b // 16, vbody, i16, unroll=8)

            @pl.when(ci < count)
            def _():
                pltpu.async_copy(
                    vbufs[p], out_hbm.at[pl.ds(chunk_eb(ci), _BLK3)], semo[p]
                )

        stage_a(jnp.int32(0), 0)

        def step(st, _):
            for p in (0, 1):
                ci = 2 * st + p
                stage_a(ci + 1, 1 - p)
                stage_b(ci, p)
            return 0

        lax.fori_loop(0, nch_max // 2, step, 0)
        pltpu.make_async_copy(accum.at[idxs[0]], sums[0], semg[0]).wait()

        @pl.when(nch_max - 1 < count)
        def _():
            pltpu.make_async_copy(
                vbufs[1], out_hbm.at[pl.ds(chunk_eb(0), _BLK3)], semo[1]
            ).wait()

    return k3


def kernel(edge_val, edge_index):
    e, h = edge_val.shape
    nb = e // 128
    ev3 = edge_val.reshape(nb, 128, h).transpose(0, 2, 1)
    row2 = edge_index[0].reshape(nb, 128)
    zeros = jnp.zeros((N_NODES, 8), jnp.float32)

    partials, wmax = _sc_scatter(nb, scaled=False)(ev3, row2, zeros)
    m8 = jnp.max(wmax.reshape(-1, 8), axis=0)
    scale16 = jnp.tile(_halving_scale(m8), 2)

    partials = lax.cond(
        jnp.any(m8 > 10.0),
        lambda: _sc_scatter(nb, scaled=True)(ev3, row2, scale16, zeros)[0],
        lambda: partials,
    )
    out3 = _sc_gather_div(nb)(ev3, row2, scale16, partials)
    return out3.transpose(0, 2, 1).reshape(e, h)


# speedup vs baseline: 353.4924x; 1.7911x over previous
import functools

import jax
import jax.numpy as jnp
from jax import lax
from jax.experimental import pallas as pl
from jax.experimental.pallas import tpu as pltpu
from jax.experimental.pallas import tpu_sc as plsc

N_NODES = 100000
N_HEADS = 8

_NC = 2
_NS = 16
_NW = _NC * _NS

_BLK2 = 16
_BLK3 = 8

_ROWS_T = (N_NODES // _NS) // 8 * 8
_TAIL = N_NODES - _ROWS_T * _NS
_SUBS = (776,) * 7 + (816,)
_SUBMAX = max(_SUBS)

_INTERPRET = False

_PARAMS = pltpu.CompilerParams(
    needs_layout_passes=False, use_tc_tiling_on_sc=False
)
_MESH = plsc.VectorSubcoreMesh(core_axis_name="c", subcore_axis_name="s")


def _halving_scale(m8):
    def cond(c):
        return jnp.any(c[0] > 10.0)

    def body(c):
        m, s = c
        h = m > 10.0
        return jnp.where(h, m * 0.5, m), jnp.where(h, s * 0.5, s)

    _, scale = lax.while_loop(cond, body, (m8, jnp.ones_like(m8)))
    return scale


def _chunk_bounds(wid, nch_total):
    first = wid * nch_total // _NW
    count = (wid + 1) * nch_total // _NW - first
    return first, count


def _copy_indices(rbuf, idxbuf, nblk):
    def ibody(j):
        idxbuf[pl.ds(j * 16, 16)] = rbuf[j >> 3, pl.ds((j & 7) * 16, 16)]

    plsc.parallel_loop(0, nblk * 8, unroll=8)(ibody)


def _sc_scatter(nb, scaled):
    nch_total = nb // _BLK2
    nch_max = -(-nch_total // _NW)
    assert nch_max % 2 == 0
    b = _BLK2 * 128

    out_type = [jax.ShapeDtypeStruct((_NC, N_NODES, 8), jnp.float32)]
    if not scaled:
        out_type.append(jax.ShapeDtypeStruct((_NC, _NS, 16), jnp.float32))

    @functools.partial(
        pl.kernel,
        out_type=out_type,
        mesh=_MESH,
        scratch_types=[
            pltpu.VMEM((_BLK2, 8, 128), jnp.float32),
            pltpu.VMEM((_BLK2, 128), jnp.int32),
            pltpu.VMEM((b, 8), jnp.float32),
            pltpu.VMEM((b, 8), jnp.float32),
            pltpu.VMEM((b,), jnp.int32),
            pltpu.VMEM((b,), jnp.int32),
            pltpu.VMEM((16,), jnp.float32),
            pltpu.VMEM_SHARED((N_NODES, 8), jnp.float32),
            pltpu.SemaphoreType.DMA,
            pltpu.SemaphoreType.DMA,
        ],
        compiler_params=_PARAMS,
        interpret=_INTERPRET,
    )
    def k2(ev_hbm, row_hbm, *args):
        if scaled:
            (scale_hbm, zeros_hbm, partials_hbm, vbuf, rbuf, ubuf0, ubuf1,
             idx0, idx1, sbuf, accum, sem0, sem1) = args
        else:
            (zeros_hbm, partials_hbm, max_hbm, vbuf, rbuf, ubuf0, ubuf1,
             idx0, idx1, sbuf, accum, sem0, sem1) = args
        ubufs, idxs, sems = (ubuf0, ubuf1), (idx0, idx1), (sem0, sem1)
        c = lax.axis_index("c")
        s = lax.axis_index("s")
        wid = c * _NS + s

        @pl.when(s == 0)
        def _():
            pltpu.sync_copy(zeros_hbm, accum)

        if scaled:
            pltpu.sync_copy(scale_hbm, sbuf)
        plsc.subcore_barrier()

        first, count = _chunk_bounds(wid, nch_total)
        i16 = lax.iota(jnp.int32, 16)
        if scaled:
            svs = [
                plsc.load_gather(sbuf, [jnp.full((16,), h, jnp.int32)])
                for h in range(8)
            ]

        def step(st, macc):
            for p in (0, 1):
                ci = 2 * st + p
                cidx = jnp.minimum(first + ci, nch_total - 1)
                eb = pl.multiple_of(cidx * _BLK2, 8)
                pltpu.sync_copy(ev_hbm.at[pl.ds(eb, _BLK2)], vbuf)
                pltpu.sync_copy(row_hbm.at[pl.ds(eb, _BLK2)], rbuf)

                @pl.when(ci >= 2)
                def _():
                    pltpu.make_async_copy(
                        ubufs[p], accum.at[idxs[p]], sems[p]
                    ).wait()

                _copy_indices(rbuf, idxs[p], _BLK2)

                for h in range(8):
                    i1 = jnp.full((16,), h, jnp.int32)

                    def vbody(t, carry):
                        m, i0 = carry
                        v = vbuf[t >> 3, h, pl.ds((t & 7) * 16, 16)]
                        if scaled:
                            e = jnp.exp(v * svs[h])
                        else:
                            e = jnp.exp(v)
                            m = jnp.maximum(m, v)
                        plsc.store_scatter(ubufs[p], [i0, i1], e)
                        return m, i0 + 16

                    macc, _ = plsc.parallel_loop(
                        0, b // 16, unroll=8, carry=(macc, i16)
                    )(vbody)

                @pl.when(ci < count)
                def _():
                    pltpu.async_copy(
                        ubufs[p], accum.at[idxs[p]], sems[p], add=True
                    )

            return macc

        macc = lax.fori_loop(
            0, nch_max // 2, step, jnp.full((16,), -jnp.inf, jnp.float32)
        )
        for p in (0, 1):
            @pl.when(nch_max - 2 + p < count)
            def _():
                pltpu.make_async_copy(
                    ubufs[p], accum.at[idxs[p]], sems[p]
                ).wait()

        if not scaled:
            sbuf[...] = macc
            pltpu.sync_copy(sbuf, max_hbm.at[c, s])
        plsc.subcore_barrier()

        @pl.when(s == 0)
        def _():
            pltpu.sync_copy(accum, partials_hbm.at[c])

    return k2


def _sc_gather_div(nb):
    nch_total = nb // _BLK3
    nch_max = -(-nch_total // _NW)
    b = _BLK3 * 128

    @functools.partial(
        pl.kernel,
        out_type=jax.ShapeDtypeStruct((nb, 8, 128), jnp.float32),
        mesh=_MESH,
        scratch_types=[
            pltpu.VMEM((_BLK3, 8, 128), jnp.float32),
            pltpu.VMEM((_BLK3, 8, 128), jnp.float32),
            pltpu.VMEM((_BLK3, 128), jnp.int32),
            pltpu.VMEM((b, 8), jnp.float32),
            pltpu.VMEM((b, 8), jnp.float32),
            pltpu.VMEM((b,), jnp.int32),
            pltpu.VMEM((b,), jnp.int32),
            pltpu.VMEM((16,), jnp.float32),
            pltpu.VMEM((_SUBMAX, 8), jnp.float32),
            pltpu.VMEM((_SUBMAX, 8), jnp.float32),
            pltpu.VMEM_SHARED((N_NODES, 8), jnp.float32),
            pltpu.SemaphoreType.DMA,
            pltpu.SemaphoreType.DMA,
            pltpu.SemaphoreType.DMA,
            pltpu.SemaphoreType.DMA,
        ],
        compiler_params=_PARAMS,
        interpret=_INTERPRET,
    )
    def k3(ev_hbm, row_hbm, scale_hbm, partials_hbm, out_hbm, vbuf0, vbuf1,
           rbuf, sums0, sums1, idx0, idx1, sbuf, pa, pb, accum,
           semg0, semg1, semo0, semo1):
        vbufs, sums, idxs = (vbuf0, vbuf1), (sums0, sums1), (idx0, idx1)
        semg, semo = (semg0, semg1), (semo0, semo1)
        c = lax.axis_index("c")
        s = lax.axis_index("s")
        wid = c * _NS + s
        i16 = lax.iota(jnp.int32, 16)
        pat0, pat1 = i16 >> 3, i16 & 7

        def combine(lo, rows):
            lo = pl.multiple_of(lo, 8)
            pltpu.sync_copy(partials_hbm.at[0, pl.ds(lo, rows)],
                            pa.at[pl.ds(0, rows)])
            pltpu.sync_copy(partials_hbm.at[1, pl.ds(lo, rows)],
                            pb.at[pl.ds(0, rows)])

            def abody(k, carry):
                i0 = carry
                v = (plsc.load_gather(pa, [i0, pat1])
                     + plsc.load_gather(pb, [i0, pat1]))
                plsc.store_scatter(pa, [i0, pat1], 1.0 / v)
                return i0 + 2

            lax.fori_loop(0, rows // 2, abody, pat0, unroll=8)
            pltpu.sync_copy(pa.at[pl.ds(0, rows)], accum.at[pl.ds(lo, rows)])

        off = 0
        for sub in _SUBS:
            combine(s * _ROWS_T + off, sub)
            off += sub

        @pl.when(s == 0)
        def _():
            combine(_ROWS_T * _NS, _TAIL)

        pltpu.sync_copy(scale_hbm, sbuf)
        plsc.subcore_barrier()

        first, count = _chunk_bounds(wid, nch_total)
        svs = [
            plsc.load_gather(sbuf, [jnp.full((16,), h, jnp.int32)])
            for h in range(8)
        ]

        def chunk_eb(ci):
            cidx = jnp.minimum(first + ci, nch_total - 1)
            return pl.multiple_of(cidx * _BLK3, 8)

        def stage_a(ci, p):
            eb = chunk_eb(ci)

            @pl.when(ci >= 2)
            def _():
                pltpu.make_async_copy(
                    vbufs[p], out_hbm.at[pl.ds(chunk_eb(ci - 2), _BLK3)],
                    semo[p],
                ).wait()

            pltpu.sync_copy(ev_hbm.at[pl.ds(eb, _BLK3)], vbufs[p])
            pltpu.sync_copy(row_hbm.at[pl.ds(eb, _BLK3)], rbuf)
            _copy_indices(rbuf, idxs[p], _BLK3)
            pltpu.async_copy(accum.at[idxs[p]], sums[p], semg[p])

        def stage_b(ci, p):
            pltpu.make_async_copy(accum.at[idxs[p]], sums[p], semg[p]).wait()

            for h in range(8):
                i1 = jnp.full((16,), h, jnp.int32)

                def vbody(t, i0):
                    sl = pl.ds((t & 7) * 16, 16)
                    v = vbufs[p][t >> 3, h, sl]
                    ns = plsc.load_gather(sums[p], [i0, i1])
                    vbufs[p][t >> 3, h, sl] = jnp.exp(v * svs[h]) * ns
                    return i0 + 16

                plsc.parallel_loop(0, b // 16, unroll=8, carry=i16)(vbody)

            @pl.when(ci < count)
            def _():
                pltpu.async_copy(
                    vbufs[p], out_hbm.at[pl.ds(chunk_eb(ci), _BLK3)], semo[p]
                )

        stage_a(jnp.int32(0), 0)

        def step(st, _):
            for p in (0, 1):
                ci = 2 * st + p
                stage_a(ci + 1, 1 - p)
                stage_b(ci, p)
            return 0

        lax.fori_loop(0, nch_max // 2, step, 0)
        pltpu.make_async_copy(accum.at[idxs[0]], sums[0], semg[0]).wait()

        @pl.when(nch_max - 1 < count)
        def _():
            pltpu.make_async_copy(
                vbufs[1], out_hbm.at[pl.ds(chunk_eb(0), _BLK3)], semo[1]
            ).wait()

    return k3


def kernel(edge_val, edge_index):
    e, h = edge_val.shape
    nb = e // 128
    ev3 = edge_val.reshape(nb, 128, h).transpose(0, 2, 1)
    row2 = edge_index[0].reshape(nb, 128)
    zeros = jnp.zeros((N_NODES, 8), jnp.float32)

    partials, wmax = _sc_scatter(nb, scaled=False)(ev3, row2, zeros)
    m8 = jnp.max(wmax.reshape(-1, 8), axis=0)
    scale16 = jnp.tile(_halving_scale(m8), 2)

    partials = lax.cond(
        jnp.any(m8 > 10.0),
        lambda: _sc_scatter(nb, scaled=True)(ev3, row2, scale16, zeros)[0],
        lambda: partials,
    )
    out3 = _sc_gather_div(nb)(ev3, row2, scale16, partials)
    return out3.transpose(0, 2, 1).reshape(e, h)


# speedup vs baseline: 361.5706x; 1.0229x over previous
import functools

import jax
import jax.numpy as jnp
from jax import lax
from jax.experimental import pallas as pl
from jax.experimental.pallas import tpu as pltpu
from jax.experimental.pallas import tpu_sc as plsc

N_NODES = 100000
N_HEADS = 8

_NC = 2
_NS = 16
_NW = _NC * _NS

_BLK2 = 16
_BLK3 = 8

_ROWS_T = (N_NODES // _NS) // 8 * 8
_TAIL = N_NODES - _ROWS_T * _NS
_SUBS = (776,) * 7 + (816,)
_SUBMAX = max(_SUBS)

_INTERPRET = False

_PARAMS = pltpu.CompilerParams(
    needs_layout_passes=False, use_tc_tiling_on_sc=False
)
_MESH = plsc.VectorSubcoreMesh(core_axis_name="c", subcore_axis_name="s")


def _halving_scale(m8):
    def cond(c):
        return jnp.any(c[0] > 10.0)

    def body(c):
        m, s = c
        h = m > 10.0
        return jnp.where(h, m * 0.5, m), jnp.where(h, s * 0.5, s)

    _, scale = lax.while_loop(cond, body, (m8, jnp.ones_like(m8)))
    return scale


def _chunk_bounds(wid, nch_total):
    first = wid * nch_total // _NW
    count = (wid + 1) * nch_total // _NW - first
    return first, count


def _copy_indices(rbuf, idxbuf, nblk):
    def ibody(j):
        idxbuf[pl.ds(j * 16, 16)] = rbuf[j >> 3, pl.ds((j & 7) * 16, 16)]

    plsc.parallel_loop(0, nblk * 8, unroll=8)(ibody)


def _sc_scatter(nb, scaled):
    nch_total = nb // _BLK2
    nch_max = -(-nch_total // _NW)
    assert nch_max % 2 == 0
    b = _BLK2 * 128

    out_type = [jax.ShapeDtypeStruct((_NC, N_NODES, 8), jnp.float32)]
    if not scaled:
        out_type.append(jax.ShapeDtypeStruct((_NC, _NS, 16), jnp.float32))

    @functools.partial(
        pl.kernel,
        out_type=out_type,
        mesh=_MESH,
        scratch_types=[
            pltpu.VMEM((_BLK2, 8, 128), jnp.float32),
            pltpu.VMEM((_BLK2, 128), jnp.int32),
            pltpu.VMEM((b, 8), jnp.float32),
            pltpu.VMEM((b, 8), jnp.float32),
            pltpu.VMEM((b,), jnp.int32),
            pltpu.VMEM((b,), jnp.int32),
            pltpu.VMEM((16,), jnp.float32),
            pltpu.VMEM_SHARED((N_NODES, 8), jnp.float32),
            pltpu.SemaphoreType.DMA,
            pltpu.SemaphoreType.DMA,
        ],
        compiler_params=_PARAMS,
        interpret=_INTERPRET,
    )
    def k2(ev_hbm, row_hbm, *args):
        if scaled:
            (scale_hbm, zeros_hbm, partials_hbm, vbuf, rbuf, ubuf0, ubuf1,
             idx0, idx1, sbuf, accum, sem0, sem1) = args
        else:
            (zeros_hbm, partials_hbm, max_hbm, vbuf, rbuf, ubuf0, ubuf1,
             idx0, idx1, sbuf, accum, sem0, sem1) = args
        ubufs, idxs, sems = (ubuf0, ubuf1), (idx0, idx1), (sem0, sem1)
        c = lax.axis_index("c")
        s = lax.axis_index("s")
        wid = c * _NS + s

        @pl.when(s == 0)
        def _():
            pltpu.sync_copy(zeros_hbm, accum)

        if scaled:
            pltpu.sync_copy(scale_hbm, sbuf)
        plsc.subcore_barrier()

        first, count = _chunk_bounds(wid, nch_total)
        i16 = lax.iota(jnp.int32, 16)
        if scaled:
            svs = [
                plsc.load_gather(sbuf, [jnp.full((16,), h, jnp.int32)])
                for h in range(8)
            ]

        def step(st, macc):
            for p in (0, 1):
                ci = 2 * st + p
                cidx = jnp.minimum(first + ci, nch_total - 1)
                eb = pl.multiple_of(cidx * _BLK2, 8)
                pltpu.sync_copy(ev_hbm.at[pl.ds(eb, _BLK2)], vbuf)
                pltpu.sync_copy(row_hbm.at[pl.ds(eb, _BLK2)], rbuf)

                @pl.when(ci >= 2)
                def _():
                    pltpu.make_async_copy(
                        ubufs[p], accum.at[idxs[p]], sems[p]
                    ).wait()

                _copy_indices(rbuf, idxs[p], _BLK2)

                for h in range(8):
                    i1 = jnp.full((16,), h, jnp.int32)

                    def vbody(t, carry):
                        m, i0 = carry
                        v = vbuf[t >> 3, h, pl.ds((t & 7) * 16, 16)]
                        if scaled:
                            e = jnp.exp(v * svs[h])
                        else:
                            e = jnp.exp(v)
                            m = jnp.maximum(m, v)
                        plsc.store_scatter(ubufs[p], [i0, i1], e)
                        return m, i0 + 16

                    macc, _ = plsc.parallel_loop(
                        0, b // 16, unroll=8, carry=(macc, i16)
                    )(vbody)

                @pl.when(ci < count)
                def _():
                    pltpu.async_copy(
                        ubufs[p], accum.at[idxs[p]], sems[p], add=True
                    )

            return macc

        macc = lax.fori_loop(
            0, nch_max // 2, step, jnp.full((16,), -jnp.inf, jnp.float32)
        )
        for p in (0, 1):
            @pl.when(nch_max - 2 + p < count)
            def _():
                pltpu.make_async_copy(
                    ubufs[p], accum.at[idxs[p]], sems[p]
                ).wait()

        if not scaled:
            sbuf[...] = macc
            pltpu.sync_copy(sbuf, max_hbm.at[c, s])
        plsc.subcore_barrier()

        @pl.when(s == 0)
        def _():
            pltpu.sync_copy(accum, partials_hbm.at[c])

    return k2


def _sc_gather_div(nb):
    nch_total = nb // _BLK3
    nch_max = -(-nch_total // _NW)
    b = _BLK3 * 128

    @functools.partial(
        pl.kernel,
        out_type=jax.ShapeDtypeStruct((nb, 8, 128), jnp.float32),
        mesh=_MESH,
        scratch_types=[
            pltpu.VMEM((_BLK3, 8, 128), jnp.float32),
            pltpu.VMEM((_BLK3, 8, 128), jnp.float32),
            pltpu.VMEM((_BLK3, 128), jnp.int32),
            pltpu.VMEM((b, 8), jnp.float32),
            pltpu.VMEM((b, 8), jnp.float32),
            pltpu.VMEM((b,), jnp.int32),
            pltpu.VMEM((b,), jnp.int32),
            pltpu.VMEM((16,), jnp.float32),
            pltpu.VMEM((_SUBMAX, 8), jnp.float32),
            pltpu.VMEM((_SUBMAX, 8), jnp.float32),
            pltpu.VMEM_SHARED((N_NODES, 8), jnp.float32),
            pltpu.SemaphoreType.DMA,
            pltpu.SemaphoreType.DMA,
            pltpu.SemaphoreType.DMA,
            pltpu.SemaphoreType.DMA,
        ],
        compiler_params=_PARAMS,
        interpret=_INTERPRET,
    )
    def k3(ev_hbm, row_hbm, scale_hbm, partials_hbm, out_hbm, vbuf0, vbuf1,
           rbuf, sums0, sums1, idx0, idx1, sbuf, pa, pb, accum,
           semg0, semg1, semo0, semo1):
        vbufs, sums, idxs = (vbuf0, vbuf1), (sums0, sums1), (idx0, idx1)
        semg, semo = (semg0, semg1), (semo0, semo1)
        c = lax.axis_index("c")
        s = lax.axis_index("s")
        wid = c * _NS + s
        i16 = lax.iota(jnp.int32, 16)
        pat0, pat1 = i16 >> 3, i16 & 7

        def combine(lo, rows):
            lo = pl.multiple_of(lo, 8)
            pltpu.sync_copy(partials_hbm.at[0, pl.ds(lo, rows)],
                            pa.at[pl.ds(0, rows)])
            pltpu.sync_copy(partials_hbm.at[1, pl.ds(lo, rows)],
                            pb.at[pl.ds(0, rows)])

            def abody(k, i0):
                v = (plsc.load_gather(pa, [i0, pat1])
                     + plsc.load_gather(pb, [i0, pat1]))
                plsc.store_scatter(pa, [i0, pat1], 1.0 / v)
                return i0 + 2

            plsc.parallel_loop(0, rows // 2, unroll=8, carry=pat0)(abody)
            pltpu.sync_copy(pa.at[pl.ds(0, rows)], accum.at[pl.ds(lo, rows)])

        off = 0
        for sub in _SUBS:
            combine(s * _ROWS_T + off, sub)
            off += sub

        @pl.when(s == 0)
        def _():
            combine(_ROWS_T * _NS, _TAIL)

        pltpu.sync_copy(scale_hbm, sbuf)
        plsc.subcore_barrier()

        first, count = _chunk_bounds(wid, nch_total)
        svs = [
            plsc.load_gather(sbuf, [jnp.full((16,), h, jnp.int32)])
            for h in range(8)
        ]

        def chunk_eb(ci):
            cidx = jnp.minimum(first + ci, nch_total - 1)
            return pl.multiple_of(cidx * _BLK3, 8)

        def stage_a(ci, p):
            eb = chunk_eb(ci)

            @pl.when(ci >= 2)
            def _():
                pltpu.make_async_copy(
                    vbufs[p], out_hbm.at[pl.ds(chunk_eb(ci - 2), _BLK3)],
                    semo[p],
                ).wait()

            pltpu.sync_copy(ev_hbm.at[pl.ds(eb, _BLK3)], vbufs[p])
            pltpu.sync_copy(row_hbm.at[pl.ds(eb, _BLK3)], rbuf)
            _copy_indices(rbuf, idxs[p], _BLK3)
            pltpu.async_copy(accum.at[idxs[p]], sums[p], semg[p])

        def stage_b(ci, p):
            pltpu.make_async_copy(accum.at[idxs[p]], sums[p], semg[p]).wait()

            for h in range(8):
                i1 = jnp.full((16,), h, jnp.int32)

                def vbody(t, i0):
                    sl = pl.ds((t & 7) * 16, 16)
                    v = vbufs[p][t >> 3, h, sl]
                    ns = plsc.load_gather(sums[p], [i0, i1])
                    vbufs[p][t >> 3, h, sl] = jnp.exp(v * svs[h]) * ns
                    return i0 + 16

                plsc.parallel_loop(0, b // 16, unroll=8, carry=i16)(vbody)

            @pl.when(ci < count)
            def _():
                pltpu.async_copy(
                    vbufs[p], out_hbm.at[pl.ds(chunk_eb(ci), _BLK3)], semo[p]
                )

        stage_a(jnp.int32(0), 0)

        def step(st, _):
            for p in (0, 1):
                ci = 2 * st + p
                stage_a(ci + 1, 1 - p)
                stage_b(ci, p)
            return 0

        lax.fori_loop(0, nch_max // 2, step, 0)
        pltpu.make_async_copy(accum.at[idxs[0]], sums[0], semg[0]).wait()

        @pl.when(nch_max - 1 < count)
        def _():
            pltpu.make_async_copy(
                vbufs[1], out_hbm.at[pl.ds(chunk_eb(0), _BLK3)], semo[1]
            ).wait()

    return k3


def kernel(edge_val, edge_index):
    e, h = edge_val.shape
    nb = e // 128
    ev3 = edge_val.reshape(nb, 128, h).transpose(0, 2, 1)
    row2 = edge_index[0].reshape(nb, 128)
    zeros = jnp.zeros((N_NODES, 8), jnp.float32)

    partials, wmax = _sc_scatter(nb, scaled=False)(ev3, row2, zeros)
    m8 = jnp.max(wmax.reshape(-1, 8), axis=0)
    scale16 = jnp.tile(_halving_scale(m8), 2)

    partials = lax.cond(
        jnp.any(m8 > 10.0),
        lambda: _sc_scatter(nb, scaled=True)(ev3, row2, scale16, zeros)[0],
        lambda: partials,
    )
    out3 = _sc_gather_div(nb)(ev3, row2, scale16, partials)
    return out3.transpose(0, 2, 1).reshape(e, h)
